# baseline (speedup 1.0000x reference)
import functools
import math

import jax
import jax.numpy as jnp
from jax.experimental import pallas as pl

Q = 1024
K = 100000
KPAD = 102400
D = 128
BK = 2048


def _sims_body(q_ref, g_ref, out_ref):
    logits = jax.lax.dot_general(
        q_ref[...], g_ref[...],
        dimension_numbers=(((1,), (1,)), ((), ())),
        preferred_element_type=jnp.float32,
    ) / jnp.sqrt(jnp.asarray(D, jnp.float32))
    out_ref[...] = jax.nn.sigmoid(logits)


@jax.jit
def _sims(q_features, g_features):
    g_pad = jnp.pad(g_features, ((0, KPAD - K), (0, 0)))
    grid = KPAD // BK
    return pl.pallas_call(
        _sims_body,
        grid=(grid,),
        in_specs=[
            pl.BlockSpec((Q, D), lambda i: (0, 0)),
            pl.BlockSpec((BK, D), lambda i: (i, 0)),
        ],
        out_specs=pl.BlockSpec((Q, BK), lambda i: (0, i)),
        out_shape=jax.ShapeDtypeStruct((Q, KPAD), jnp.float32),
    )(q_features, g_pad)


def kernel(q_features, g_features):
    sims = _sims(q_features, g_features)[:, :K]
    neg = -sims
    sims_sorted = jnp.sort(neg, axis=-1)
    pred_ranks = jnp.argsort(neg, axis=-1)
    return (sims_sorted, pred_ranks)
